# baseline (speedup 1.0000x reference)
import jax
import jax.numpy as jnp
from jax.experimental import pallas as pl
from jax.experimental.pallas import tpu as pltpu

_SHAPE = (2, 16384, 1024)
_N_STREAMS = 16
_CHUNK_ROWS = (2 * _SHAPE[1]) // _N_STREAMS
_TILE_ROWS = 8


def _copy_update_kernel(key_ref, in_ref, out_ref, tile_ref, sem_big, sem_small):
    copies = []
    for i in range(_N_STREAMS):
        c = pltpu.make_async_copy(
            in_ref.at[pl.ds(i * _CHUNK_ROWS, _CHUNK_ROWS), :],
            out_ref.at[pl.ds(i * _CHUNK_ROWS, _CHUNK_ROWS), :],
            sem_big.at[i],
        )
        c.start()
        copies.append(c)

    small_in = pltpu.make_async_copy(
        in_ref.at[pl.ds(_SHAPE[1], _TILE_ROWS), :], tile_ref, sem_small
    )
    small_in.start()
    small_in.wait()
    row = jax.lax.broadcasted_iota(jnp.int32, (_TILE_ROWS, _SHAPE[2]), 0)
    col = jax.lax.broadcasted_iota(jnp.int32, (_TILE_ROWS, _SHAPE[2]), 1)
    mask = (row == 0) & (col == 1)
    tile_ref[...] += jnp.where(mask, 2.0 * key_ref[0], 0.0)
    for c in copies:
        c.wait()
    small_out = pltpu.make_async_copy(
        tile_ref, out_ref.at[pl.ds(_SHAPE[1], _TILE_ROWS), :], sem_small
    )
    small_out.start()
    small_out.wait()


def kernel(key, cache_next):
    flat = cache_next.reshape(2 * _SHAPE[1], _SHAPE[2])
    out = pl.pallas_call(
        _copy_update_kernel,
        out_shape=jax.ShapeDtypeStruct((2 * _SHAPE[1], _SHAPE[2]), jnp.float32),
        in_specs=[
            pl.BlockSpec(memory_space=pltpu.SMEM),
            pl.BlockSpec(memory_space=pl.ANY),
        ],
        out_specs=pl.BlockSpec(memory_space=pl.ANY),
        scratch_shapes=[
            pltpu.VMEM((_TILE_ROWS, _SHAPE[2]), jnp.float32),
            pltpu.SemaphoreType.DMA((_N_STREAMS,)),
            pltpu.SemaphoreType.DMA,
        ],
    )(key, flat)
    return key, out.reshape(_SHAPE)


# speedup vs baseline: 47.8020x; 47.8020x over previous
import jax
import jax.numpy as jnp
from jax.experimental import pallas as pl
from jax.experimental.pallas import tpu as pltpu

_SHAPE = (2, 16384, 1024)
_FLAT_ROWS = 2 * _SHAPE[1]
_N_CHUNKS = 16
_CHUNK_ROWS = _FLAT_ROWS // _N_CHUNKS
_NBUF = 4
_PATCH_CHUNK = _SHAPE[1] // _CHUNK_ROWS


def _copy_update_kernel(key_ref, in_ref, out_ref, bufs, sem_in, sem_out):
    def start_in(i):
        pltpu.make_async_copy(
            in_ref.at[pl.ds(i * _CHUNK_ROWS, _CHUNK_ROWS), :],
            bufs.at[i % _NBUF],
            sem_in.at[i % _NBUF],
        ).start()

    def wait_in(i):
        pltpu.make_async_copy(
            in_ref.at[pl.ds(i * _CHUNK_ROWS, _CHUNK_ROWS), :],
            bufs.at[i % _NBUF],
            sem_in.at[i % _NBUF],
        ).wait()

    def start_out(i):
        pltpu.make_async_copy(
            bufs.at[i % _NBUF],
            out_ref.at[pl.ds(i * _CHUNK_ROWS, _CHUNK_ROWS), :],
            sem_out.at[i % _NBUF],
        ).start()

    def wait_out(i):
        pltpu.make_async_copy(
            bufs.at[i % _NBUF],
            out_ref.at[pl.ds(i * _CHUNK_ROWS, _CHUNK_ROWS), :],
            sem_out.at[i % _NBUF],
        ).wait()

    lookahead = _NBUF // 2
    for i in range(lookahead):
        start_in(i)
    for i in range(_N_CHUNKS):
        nxt = i + lookahead
        if nxt < _N_CHUNKS:
            if nxt >= _NBUF:
                wait_out(nxt - _NBUF)
            start_in(nxt)
        wait_in(i)
        if i == _PATCH_CHUNK:
            row = jax.lax.broadcasted_iota(jnp.int32, (8, 128), 0)
            col = jax.lax.broadcasted_iota(jnp.int32, (8, 128), 1)
            mask = (row == 0) & (col == 1)
            bufs[i % _NBUF, 0:8, 0:128] += jnp.where(
                mask, 2.0 * key_ref[0], 0.0
            )
        start_out(i)
    for i in range(_N_CHUNKS - _NBUF, _N_CHUNKS):
        wait_out(i)


def kernel(key, cache_next):
    flat = cache_next.reshape(_FLAT_ROWS, _SHAPE[2])
    out = pl.pallas_call(
        _copy_update_kernel,
        out_shape=jax.ShapeDtypeStruct((_FLAT_ROWS, _SHAPE[2]), jnp.float32),
        in_specs=[
            pl.BlockSpec(memory_space=pltpu.SMEM),
            pl.BlockSpec(memory_space=pl.ANY),
        ],
        out_specs=pl.BlockSpec(memory_space=pl.ANY),
        scratch_shapes=[
            pltpu.VMEM((_NBUF, _CHUNK_ROWS, _SHAPE[2]), jnp.float32),
            pltpu.SemaphoreType.DMA((_NBUF,)),
            pltpu.SemaphoreType.DMA((_NBUF,)),
        ],
    )(key, flat)
    return key, out.reshape(_SHAPE)
